# Initial kernel scaffold; baseline (speedup 1.0000x reference)
import jax
import jax.numpy as jnp
from jax.experimental import pallas as pl


def kernel(x, pos, pe_init, exW1, exb1, exW2, exb2, epW, epb, mW1, mb1, mW2, mb2, eW, eb, unW1, unb1, unW2, unb2, upW1, upb1, upW2, upb2, ndW1, ndb1, ndW2, ndb2, hW1, hb1, hW2, hb2, edge_index, batch):
    raise NotImplementedError("write your pallas kernel here")


# speedup vs baseline: 3.8583x; 3.8583x over previous
import dataclasses
import functools

import jax
import jax.numpy as jnp
from jax import lax
from jax.experimental import pallas as pl
from jax.experimental.pallas import tpu as pltpu
from jax.experimental.pallas import tpu_sc as plsc

_N = 10000
_E = 160000
_H = 128
_NG = 64

_W = 128
_NB = _E // _W
_NBC = _NB // 2
_NP = 10112
_RPT = _NP // 16
_CHUNK = _E // 32
_RB = 400
_GN = _N // _RB
_EB = 640
_GE = _E // _EB


def _vmesh():
    return plsc.VectorSubcoreMesh(core_axis_name="core", subcore_axis_name="subcore")


def _sc_params():
    cp = pltpu.CompilerParams()
    if "needs_layout_passes" in pltpu.CompilerParams.__dataclass_fields__:
        cp = dataclasses.replace(cp, needs_layout_passes=False)
    return cp


def _sc_sqdist_deg(px, py, pz, dst, src):
    ngrp = _CHUNK // 16 + 1

    @functools.partial(
        pl.kernel,
        out_type=(
            jax.ShapeDtypeStruct((_E,), jnp.float32),
            jax.ShapeDtypeStruct((32, _N), jnp.float32),
        ),
        mesh=_vmesh(),
        scratch_types=[
            pltpu.VMEM((_N,), jnp.float32),
            pltpu.VMEM((_N,), jnp.float32),
            pltpu.VMEM((_N,), jnp.float32),
            pltpu.VMEM((_CHUNK + 16,), jnp.int32),
            pltpu.VMEM((_CHUNK + 16,), jnp.int32),
            pltpu.VMEM((_CHUNK + 16,), jnp.float32),
            pltpu.VMEM((_N,), jnp.float32),
        ],
        compiler_params=_sc_params(),
    )
    def k(px_hbm, py_hbm, pz_hbm, d_hbm, s_hbm, o_hbm, deg_hbm,
          px_v, py_v, pz_v, d_v, s_v, o_v, deg_v):
        cid = lax.axis_index("core")
        sid = lax.axis_index("subcore")
        wid = sid * 2 + cid
        base = wid * _CHUNK
        zero16 = jnp.zeros((16,), jnp.int32)
        zero16f = jnp.zeros((16,), jnp.float32)
        one16f = jnp.ones((16,), jnp.float32)
        lane = lax.iota(jnp.int32, 16)
        d_v[pl.ds(_CHUNK - 8, 16)] = zero16
        s_v[pl.ds(_CHUNK - 8, 16)] = zero16

        @pl.loop(0, _N, step=16)
        def _(r):
            deg_v[pl.ds(r, 16)] = zero16f

        pltpu.sync_copy(px_hbm, px_v)
        pltpu.sync_copy(py_hbm, py_v)
        pltpu.sync_copy(pz_hbm, pz_v)
        pltpu.sync_copy(d_hbm.at[pl.ds(base, _CHUNK)], d_v.at[pl.ds(0, _CHUNK)])
        pltpu.sync_copy(s_hbm.at[pl.ds(base, _CHUNK)], s_v.at[pl.ds(0, _CHUNK)])

        @pl.loop(0, ngrp)
        def _(g):
            off = g * 16
            di = d_v[pl.ds(off, 16)]
            si = s_v[pl.ds(off, 16)]
            dx = plsc.load_gather(px_v, [di]) - plsc.load_gather(px_v, [si])
            dy = plsc.load_gather(py_v, [di]) - plsc.load_gather(py_v, [si])
            dz = plsc.load_gather(pz_v, [di]) - plsc.load_gather(pz_v, [si])
            o_v[pl.ds(off, 16)] = dx * dx + dy * dy + dz * dz
            ones = jnp.where(off + lane < _CHUNK, one16f, zero16f)
            plsc.addupdate_scatter(deg_v, [di], ones)

        pltpu.sync_copy(o_v.at[pl.ds(0, _CHUNK)], o_hbm.at[pl.ds(base, _CHUNK)])
        pltpu.sync_copy(deg_v, deg_hbm.at[wid])

    return k(px, py, pz, dst, src)


def _sc_gather_pre(ad, as_, dst2d, src2d):
    @functools.partial(
        pl.kernel,
        out_type=jax.ShapeDtypeStruct((_E, _H), jnp.float32),
        mesh=_vmesh(),
        scratch_types=[
            pltpu.VMEM((_W, _H), jnp.float32),
            pltpu.SemaphoreType.DMA,
            pltpu.SemaphoreType.DMA,
        ],
    )
    def k(ad_hbm, as_hbm, d_hbm, s_hbm, o_hbm, tmp_v, sem_a, sem_b):
        def body(d_v, s_v, o_v):
            ca = pltpu.async_copy(ad_hbm.at[d_v.at[0]], o_v, sem_a)
            cb = pltpu.async_copy(as_hbm.at[s_v.at[0]], tmp_v, sem_b)
            ca.wait()
            cb.wait()

            @pl.loop(0, _W)
            def _(r):
                @pl.loop(0, _H, step=16)
                def _(c):
                    o_v[r, pl.ds(c, 16)] += tmp_v[r, pl.ds(c, 16)]

        pltpu.emit_pipeline(
            body,
            grid=(_NB,),
            in_specs=[
                pl.BlockSpec((1, _W), lambda i: (i, 0)),
                pl.BlockSpec((1, _W), lambda i: (i, 0)),
            ],
            out_specs=[pl.BlockSpec((_W, _H), lambda i: (i, 0))],
            core_axis_name=("core", "subcore"),
            dimension_semantics=(pltpu.PARALLEL,),
        )(d_hbm, s_hbm, o_hbm)

    return k(ad, as_, dst2d, src2d)


def _sc_scatter(msg, dst2d, zrow):
    npt = _NBC // 16 + 1

    @functools.partial(
        pl.kernel,
        out_type=jax.ShapeDtypeStruct((2, _NP, _H), jnp.float32),
        mesh=_vmesh(),
        scratch_types=[
            pltpu.VMEM_SHARED((_NP, _H), jnp.float32),
            pltpu.VMEM((_W, _H), jnp.float32),
            pltpu.VMEM((_W,), jnp.int32),
        ],
    )
    def k(msg_hbm, d_hbm, z_hbm, agg_hbm, shared, m_v, i_v):
        cid = lax.axis_index("core")
        sid = lax.axis_index("subcore")
        row0 = pl.multiple_of(sid * _RPT, 8)
        pltpu.sync_copy(z_hbm, shared.at[pl.ds(row0, _RPT)])
        plsc.subcore_barrier()

        @pl.loop(0, npt)
        def _(j):
            b = sid + j * 16

            @pl.when(b < _NBC)
            def _():
                blk = cid * _NBC + b
                pltpu.sync_copy(d_hbm.at[blk], i_v)
                pltpu.sync_copy(msg_hbm.at[pl.ds(blk * _W, _W)], m_v)
                pltpu.sync_copy(m_v, shared.at[i_v], add=True)

        plsc.subcore_barrier()
        pltpu.sync_copy(shared.at[pl.ds(row0, _RPT)],
                        agg_hbm.at[cid].at[pl.ds(row0, _RPT)])

    return k(msg, dst2d, zrow)


def _full_spec(shape):
    n = len(shape)
    return pl.BlockSpec(shape, lambda i, _n=n: (0,) * _n)


def _row_spec():
    return pl.BlockSpec((_RB, _H), lambda i: (i, 0))


def _tc_embed(x, pe0p, exW1, exb1, exW2, exb2, epWp, epb, wd, wpd, ws, wps, bm):
    def body(x_ref, pe0_ref, w1, b1, w2, b2, wp, bp, wdr, wpdr, wsr, wpsr, bmr,
             h_ref, pe_ref, ad_ref, as_ref):
        h1 = jnp.maximum(x_ref[...] @ w1[...] + b1[...], 0.0)
        h = h1 @ w2[...] + b2[...]
        pe = pe0_ref[...] @ wp[...] + bp[...]
        h_ref[...] = h
        pe_ref[...] = pe
        ad_ref[...] = h @ wdr[...] + pe @ wpdr[...] + bmr[...]
        as_ref[...] = h @ wsr[...] + pe @ wpsr[...]

    args = (x, pe0p, exW1, exb1, exW2, exb2, epWp, epb, wd, wpd, ws, wps, bm)
    out = jax.ShapeDtypeStruct((_N, _H), jnp.float32)
    return pl.pallas_call(
        body,
        grid=(_GN,),
        in_specs=[_row_spec(), _row_spec()] + [_full_spec(a.shape) for a in args[2:]],
        out_specs=[_row_spec()] * 4,
        out_shape=[out] * 4,
    )(*args)


def _tc_edge_mlp(pre, sq3d, w2, b2, ew_row, eb11, wdist_row):
    def body(pre_ref, sq_ref, w2r, b2r, ewr, ebr, wdr, out_ref):
        dist = jnp.sqrt(sq_ref[0, 0, :])
        p = pre_ref[...] + dist[:, None] * wdr[...]
        m = jnp.maximum(jnp.maximum(p, 0.0) @ w2r[...] + b2r[...], 0.0)
        g = jax.nn.sigmoid(jnp.sum(m * ewr[...], axis=1, keepdims=True) + ebr[...])
        out_ref[...] = m * g

    args = (pre, sq3d, w2, b2, ew_row, eb11, wdist_row)
    return pl.pallas_call(
        body,
        grid=(_GE,),
        in_specs=[
            pl.BlockSpec((_EB, _H), lambda i: (i, 0)),
            pl.BlockSpec((1, 1, _EB), lambda i: (i, 0, 0)),
        ] + [_full_spec(a.shape) for a in args[2:]],
        out_specs=pl.BlockSpec((_EB, _H), lambda i: (i, 0)),
        out_shape=jax.ShapeDtypeStruct((_E, _H), jnp.float32),
    )(*args)


def _tc_deg_inv(degp):
    def body(d_ref, out_ref):
        cnt = jnp.sum(d_ref[...], axis=0)
        out_ref[...] = (1.0 / jnp.maximum(cnt, 1.0))[:, None]

    return pl.pallas_call(
        body,
        grid=(1,),
        in_specs=[_full_spec(degp.shape)],
        out_specs=pl.BlockSpec((_N, 1), lambda i: (0, 0)),
        out_shape=jax.ShapeDtypeStruct((_N, 1), jnp.float32),
    )(degp)


def _agg_specs():
    return [
        pl.BlockSpec((2, _RB, _H), lambda i: (0, i, 0)),
        pl.BlockSpec((_RB, 1), lambda i: (i, 0)),
    ]


def _tc_update0(h, pe, aggp, degp, w1h, w1p, w1a, b1, w2, b2,
                q1p, q1a, qb1, q2, qb2, wd2, wpd2, ws2, wps2, bm2):
    def body(h_ref, pe_ref, agg_ref, deg_ref, w1hr, w1pr, w1ar, b1r, w2r, b2r,
             q1pr, q1ar, qb1r, q2r, qb2r, wdr, wpdr, wsr, wpsr, bmr,
             h_out, pe_out, ad_out, as_out):
        agg = (agg_ref[0] + agg_ref[1]) * deg_ref[...]
        h0 = h_ref[...]
        pe0 = pe_ref[...]
        u = jnp.maximum(h0 @ w1hr[...] + pe0 @ w1pr[...] + agg @ w1ar[...]
                        + b1r[...], 0.0)
        hn = h0 + u @ w2r[...] + b2r[...]
        p = jnp.maximum(pe0 @ q1pr[...] + agg[:, :_H // 2] @ q1ar[...]
                        + qb1r[...], 0.0)
        pn = pe0 + p @ q2r[...] + qb2r[...]
        h_out[...] = hn
        pe_out[...] = pn
        ad_out[...] = hn @ wdr[...] + pn @ wpdr[...] + bmr[...]
        as_out[...] = hn @ wsr[...] + pn @ wpsr[...]

    args = (h, pe, aggp, degp, w1h, w1p, w1a, b1, w2, b2,
            q1p, q1a, qb1, q2, qb2, wd2, wpd2, ws2, wps2, bm2)
    out = jax.ShapeDtypeStruct((_N, _H), jnp.float32)
    return pl.pallas_call(
        body,
        grid=(_GN,),
        in_specs=[_row_spec(), _row_spec()] + _agg_specs()
        + [_full_spec(a.shape) for a in args[4:]],
        out_specs=[_row_spec()] * 4,
        out_shape=[out] * 4,
    )(*args)


def _tc_update1(h, pe, aggp, degp, w1h, w1p, w1a, b1, w2, b2, nd1, ndb1, nd2, ndb2):
    def body(h_ref, pe_ref, agg_ref, deg_ref, w1hr, w1pr, w1ar, b1r, w2r, b2r,
             nd1r, ndb1r, nd2r, ndb2r, out_ref):
        agg = (agg_ref[0] + agg_ref[1]) * deg_ref[...]
        h0 = h_ref[...]
        u = jnp.maximum(h0 @ w1hr[...] + pe_ref[...] @ w1pr[...]
                        + agg @ w1ar[...] + b1r[...], 0.0)
        hn = h0 + u @ w2r[...] + b2r[...]
        out_ref[...] = jnp.maximum(hn @ nd1r[...] + ndb1r[...], 0.0) @ nd2r[...] \
            + ndb2r[...]

    args = (h, pe, aggp, degp, w1h, w1p, w1a, b1, w2, b2, nd1, ndb1, nd2, ndb2)
    return pl.pallas_call(
        body,
        grid=(_GN,),
        in_specs=[_row_spec(), _row_spec()] + _agg_specs()
        + [_full_spec(a.shape) for a in args[4:]],
        out_specs=_row_spec(),
        out_shape=jax.ShapeDtypeStruct((_N, _H), jnp.float32),
    )(*args)


def _tc_pool_head(nd, batch3d, hw1, hb1, hw2, hb2):
    def body(nd_ref, b_ref, w1, b1, w2, b2, out_ref, acc):
        i = pl.program_id(0)
        ids = b_ref[0, 0, :]
        oh = (ids[:, None]
              == lax.broadcasted_iota(jnp.int32, (_RB, _NG), 1)).astype(jnp.float32)
        contrib = lax.dot_general(oh, nd_ref[...], (((0,), (0,)), ((), ())),
                                  preferred_element_type=jnp.float32)

        @pl.when(i == 0)
        def _():
            acc[...] = contrib

        @pl.when(i > 0)
        def _():
            acc[...] += contrib

        @pl.when(i == _GN - 1)
        def _():
            out_ref[...] = jnp.maximum(acc[...] @ w1[...] + b1[...], 0.0) \
                @ w2[...] + b2[...]

    args = (nd, batch3d, hw1, hb1, hw2, hb2)
    return pl.pallas_call(
        body,
        grid=(_GN,),
        in_specs=[
            _row_spec(),
            pl.BlockSpec((1, 1, _RB), lambda i: (i, 0, 0)),
        ] + [_full_spec(a.shape) for a in args[2:]],
        out_specs=pl.BlockSpec((_NG, _H), lambda i: (0, 0)),
        out_shape=jax.ShapeDtypeStruct((_NG, _H), jnp.float32),
        scratch_shapes=[pltpu.VMEM((_NG, _H), jnp.float32)],
    )(*args)


def kernel(x, pos, pe_init, exW1, exb1, exW2, exb2, epW, epb, mW1, mb1, mW2,
           mb2, eW, eb, unW1, unb1, unW2, unb2, upW1, upb1, upW2, upb2, ndW1,
           ndb1, ndW2, ndb2, hW1, hb1, hW2, hb2, edge_index, batch):
    f32 = jnp.float32
    src = edge_index[0]
    dst = edge_index[1]
    dst2d = dst.reshape(_NB, _W)
    src2d = src.reshape(_NB, _W)
    posx = pos[:, 0]
    posy = pos[:, 1]
    posz = pos[:, 2]
    pe0p = jnp.pad(pe_init, ((0, 0), (0, _H - 16)))
    epWp = jnp.pad(epW, ((0, _H - 16), (0, 0)))

    def r1(b):
        return b.reshape(1, -1)

    wd = [mW1[l, 0:_H] for l in range(2)]
    ws = [mW1[l, _H:2 * _H] for l in range(2)]
    wpd = [mW1[l, 2 * _H:3 * _H] for l in range(2)]
    wps = [mW1[l, 3 * _H:4 * _H] for l in range(2)]
    wdist = [mW1[l, 4 * _H].reshape(1, _H) for l in range(2)]

    sq, degp32 = _sc_sqdist_deg(posx, posy, posz, dst, src)
    degp = _tc_deg_inv(degp32)
    sq3d = sq.reshape(_GE, 1, _EB)
    h0, pe0, ad1, as1 = _tc_embed(
        x, pe0p, exW1, r1(exb1), exW2, r1(exb2), epWp, r1(epb),
        wd[0], wpd[0], ws[0], wps[0], r1(mb1[0]))

    zrow = jnp.zeros((_RPT, _H), f32)

    pre1 = _sc_gather_pre(ad1, as1, dst2d, src2d)
    msg1 = _tc_edge_mlp(pre1, sq3d, mW2[0], r1(mb2[0]),
                        eW[0].reshape(1, _H), eb[0].reshape(1, 1), wdist[0])
    aggp1 = _sc_scatter(msg1, dst2d, zrow)
    h1, pe1, ad2, as2 = _tc_update0(
        h0, pe0, aggp1, degp,
        unW1[0, 0:_H], unW1[0, _H:2 * _H], unW1[0, 2 * _H:3 * _H],
        r1(unb1[0]), unW2[0], r1(unb2[0]),
        upW1[0, 0:_H], upW1[0, _H:_H + _H // 2], r1(upb1[0]),
        upW2[0], r1(upb2[0]),
        wd[1], wpd[1], ws[1], wps[1], r1(mb1[1]))

    pre2 = _sc_gather_pre(ad2, as2, dst2d, src2d)
    msg2 = _tc_edge_mlp(pre2, sq3d, mW2[1], r1(mb2[1]),
                        eW[1].reshape(1, _H), eb[1].reshape(1, 1), wdist[1])
    aggp2 = _sc_scatter(msg2, dst2d, zrow)
    nd = _tc_update1(
        h1, pe1, aggp2, degp,
        unW1[1, 0:_H], unW1[1, _H:2 * _H], unW1[1, 2 * _H:3 * _H],
        r1(unb1[1]), unW2[1], r1(unb2[1]),
        ndW1, r1(ndb1), ndW2, r1(ndb2))

    batch3d = batch.reshape(_GN, 1, _RB)
    return _tc_pool_head(nd, batch3d, hW1, r1(hb1), hW2, r1(hb2))


# speedup vs baseline: 5.0204x; 1.3012x over previous
import dataclasses
import functools

import jax
import jax.numpy as jnp
from jax import lax
from jax.experimental import pallas as pl
from jax.experimental.pallas import tpu as pltpu
from jax.experimental.pallas import tpu_sc as plsc

_N = 10000
_E = 160000
_H = 128
_NG = 64

_W = 128
_NB = _E // _W
_NBC = _NB // 2
_NP = 10112
_RPT = _NP // 16
_CHUNK = _E // 32
_RB = 400
_GN = _N // _RB
_EB = 640
_GE = _E // _EB


def _vmesh():
    return plsc.VectorSubcoreMesh(core_axis_name="core", subcore_axis_name="subcore")


def _sc_params():
    cp = pltpu.CompilerParams()
    if "needs_layout_passes" in pltpu.CompilerParams.__dataclass_fields__:
        cp = dataclasses.replace(cp, needs_layout_passes=False)
    return cp


def _sc_sqdist_deg(px, py, pz, dst, src):
    ngrp = _CHUNK // 16 + 1

    @functools.partial(
        pl.kernel,
        out_type=(
            jax.ShapeDtypeStruct((_E,), jnp.float32),
            jax.ShapeDtypeStruct((32, _N), jnp.float32),
        ),
        mesh=_vmesh(),
        scratch_types=[
            pltpu.VMEM((_N,), jnp.float32),
            pltpu.VMEM((_N,), jnp.float32),
            pltpu.VMEM((_N,), jnp.float32),
            pltpu.VMEM((_CHUNK + 16,), jnp.int32),
            pltpu.VMEM((_CHUNK + 16,), jnp.int32),
            pltpu.VMEM((_CHUNK + 16,), jnp.float32),
            pltpu.VMEM((_N,), jnp.float32),
        ],
        compiler_params=_sc_params(),
    )
    def k(px_hbm, py_hbm, pz_hbm, d_hbm, s_hbm, o_hbm, deg_hbm,
          px_v, py_v, pz_v, d_v, s_v, o_v, deg_v):
        cid = lax.axis_index("core")
        sid = lax.axis_index("subcore")
        wid = sid * 2 + cid
        base = wid * _CHUNK
        zero16 = jnp.zeros((16,), jnp.int32)
        zero16f = jnp.zeros((16,), jnp.float32)
        one16f = jnp.ones((16,), jnp.float32)
        lane = lax.iota(jnp.int32, 16)
        d_v[pl.ds(_CHUNK - 8, 16)] = zero16
        s_v[pl.ds(_CHUNK - 8, 16)] = zero16

        @pl.loop(0, _N, step=16)
        def _(r):
            deg_v[pl.ds(r, 16)] = zero16f

        pltpu.sync_copy(px_hbm, px_v)
        pltpu.sync_copy(py_hbm, py_v)
        pltpu.sync_copy(pz_hbm, pz_v)
        pltpu.sync_copy(d_hbm.at[pl.ds(base, _CHUNK)], d_v.at[pl.ds(0, _CHUNK)])
        pltpu.sync_copy(s_hbm.at[pl.ds(base, _CHUNK)], s_v.at[pl.ds(0, _CHUNK)])

        @pl.loop(0, ngrp)
        def _(g):
            off = g * 16
            di = d_v[pl.ds(off, 16)]
            si = s_v[pl.ds(off, 16)]
            dx = plsc.load_gather(px_v, [di]) - plsc.load_gather(px_v, [si])
            dy = plsc.load_gather(py_v, [di]) - plsc.load_gather(py_v, [si])
            dz = plsc.load_gather(pz_v, [di]) - plsc.load_gather(pz_v, [si])
            o_v[pl.ds(off, 16)] = dx * dx + dy * dy + dz * dz
            ones = jnp.where(off + lane < _CHUNK, one16f, zero16f)
            plsc.addupdate_scatter(deg_v, [di], ones)

        pltpu.sync_copy(o_v.at[pl.ds(0, _CHUNK)], o_hbm.at[pl.ds(base, _CHUNK)])
        pltpu.sync_copy(deg_v, deg_hbm.at[wid])

    return k(px, py, pz, dst, src)


def _sc_gather_pre(ad, as_, dst2d, src2d):
    @functools.partial(
        pl.kernel,
        out_type=(
            jax.ShapeDtypeStruct((_E, _H), jnp.float32),
            jax.ShapeDtypeStruct((_E, _H), jnp.float32),
        ),
        mesh=_vmesh(),
        scratch_types=[
            pltpu.SemaphoreType.DMA,
            pltpu.SemaphoreType.DMA,
        ],
    )
    def k(ad_hbm, as_hbm, d_hbm, s_hbm, o1_hbm, o2_hbm, sem_a, sem_b):
        def body(d_v, s_v, o1_v, o2_v):
            ca = pltpu.async_copy(ad_hbm.at[d_v.at[0]], o1_v, sem_a)
            cb = pltpu.async_copy(as_hbm.at[s_v.at[0]], o2_v, sem_b)
            ca.wait()
            cb.wait()

        pltpu.emit_pipeline(
            body,
            grid=(_NB,),
            in_specs=[
                pl.BlockSpec((1, _W), lambda i: (i, 0)),
                pl.BlockSpec((1, _W), lambda i: (i, 0)),
            ],
            out_specs=[
                pl.BlockSpec((_W, _H), lambda i: (i, 0)),
                pl.BlockSpec((_W, _H), lambda i: (i, 0)),
            ],
            core_axis_name=("core", "subcore"),
            dimension_semantics=(pltpu.PARALLEL,),
        )(d_hbm, s_hbm, o1_hbm, o2_hbm)

    return k(ad, as_, dst2d, src2d)


def _sc_scatter(msg, dst2d, zrow):
    npt = _NBC // 16 + 1

    @functools.partial(
        pl.kernel,
        out_type=jax.ShapeDtypeStruct((2, _NP, _H), jnp.float32),
        mesh=_vmesh(),
        scratch_types=[
            pltpu.VMEM_SHARED((_NP, _H), jnp.float32),
            pltpu.VMEM((_W, _H), jnp.float32),
            pltpu.VMEM((_W,), jnp.int32),
        ],
    )
    def k(msg_hbm, d_hbm, z_hbm, agg_hbm, shared, m_v, i_v):
        cid = lax.axis_index("core")
        sid = lax.axis_index("subcore")
        row0 = pl.multiple_of(sid * _RPT, 8)
        pltpu.sync_copy(z_hbm, shared.at[pl.ds(row0, _RPT)])
        plsc.subcore_barrier()

        @pl.loop(0, npt)
        def _(j):
            b = sid + j * 16

            @pl.when(b < _NBC)
            def _():
                blk = cid * _NBC + b
                pltpu.sync_copy(d_hbm.at[blk], i_v)
                pltpu.sync_copy(msg_hbm.at[pl.ds(blk * _W, _W)], m_v)
                pltpu.sync_copy(m_v, shared.at[i_v], add=True)

        plsc.subcore_barrier()
        pltpu.sync_copy(shared.at[pl.ds(row0, _RPT)],
                        agg_hbm.at[cid].at[pl.ds(row0, _RPT)])

    return k(msg, dst2d, zrow)


def _full_spec(shape):
    n = len(shape)
    return pl.BlockSpec(shape, lambda i, _n=n: (0,) * _n)


def _row_spec():
    return pl.BlockSpec((_RB, _H), lambda i: (i, 0))


def _tc_embed(x, pe0p, exW1, exb1, exW2, exb2, epWp, epb, wd, wpd, ws, wps, bm):
    def body(x_ref, pe0_ref, w1, b1, w2, b2, wp, bp, wdr, wpdr, wsr, wpsr, bmr,
             h_ref, pe_ref, ad_ref, as_ref):
        h1 = jnp.maximum(x_ref[...] @ w1[...] + b1[...], 0.0)
        h = h1 @ w2[...] + b2[...]
        pe = pe0_ref[...] @ wp[...] + bp[...]
        h_ref[...] = h
        pe_ref[...] = pe
        ad_ref[...] = h @ wdr[...] + pe @ wpdr[...] + bmr[...]
        as_ref[...] = h @ wsr[...] + pe @ wpsr[...]

    args = (x, pe0p, exW1, exb1, exW2, exb2, epWp, epb, wd, wpd, ws, wps, bm)
    out = jax.ShapeDtypeStruct((_N, _H), jnp.float32)
    return pl.pallas_call(
        body,
        grid=(_GN,),
        in_specs=[_row_spec(), _row_spec()] + [_full_spec(a.shape) for a in args[2:]],
        out_specs=[_row_spec()] * 4,
        out_shape=[out] * 4,
    )(*args)


def _tc_edge_mlp(adg, asg, sq3d, w2, b2, ew_row, eb11, wdist_row):
    def body(adg_ref, asg_ref, sq_ref, w2r, b2r, ewr, ebr, wdr, out_ref):
        dist = jnp.sqrt(sq_ref[0, 0, :])
        p = adg_ref[...] + asg_ref[...] + dist[:, None] * wdr[...]
        m = jnp.maximum(jnp.maximum(p, 0.0) @ w2r[...] + b2r[...], 0.0)
        g = jax.nn.sigmoid(jnp.sum(m * ewr[...], axis=1, keepdims=True) + ebr[...])
        out_ref[...] = m * g

    args = (adg, asg, sq3d, w2, b2, ew_row, eb11, wdist_row)
    return pl.pallas_call(
        body,
        grid=(_GE,),
        in_specs=[
            pl.BlockSpec((_EB, _H), lambda i: (i, 0)),
            pl.BlockSpec((_EB, _H), lambda i: (i, 0)),
            pl.BlockSpec((1, 1, _EB), lambda i: (i, 0, 0)),
        ] + [_full_spec(a.shape) for a in args[3:]],
        out_specs=pl.BlockSpec((_EB, _H), lambda i: (i, 0)),
        out_shape=jax.ShapeDtypeStruct((_E, _H), jnp.float32),
    )(*args)


def _tc_deg_inv(degp):
    def body(d_ref, out_ref):
        cnt = jnp.sum(d_ref[...], axis=0)
        out_ref[...] = (1.0 / jnp.maximum(cnt, 1.0))[:, None]

    return pl.pallas_call(
        body,
        grid=(1,),
        in_specs=[_full_spec(degp.shape)],
        out_specs=pl.BlockSpec((_N, 1), lambda i: (0, 0)),
        out_shape=jax.ShapeDtypeStruct((_N, 1), jnp.float32),
    )(degp)


def _agg_specs():
    return [
        pl.BlockSpec((2, _RB, _H), lambda i: (0, i, 0)),
        pl.BlockSpec((_RB, 1), lambda i: (i, 0)),
    ]


def _tc_update0(h, pe, aggp, degp, w1h, w1p, w1a, b1, w2, b2,
                q1p, q1a, qb1, q2, qb2, wd2, wpd2, ws2, wps2, bm2):
    def body(h_ref, pe_ref, agg_ref, deg_ref, w1hr, w1pr, w1ar, b1r, w2r, b2r,
             q1pr, q1ar, qb1r, q2r, qb2r, wdr, wpdr, wsr, wpsr, bmr,
             h_out, pe_out, ad_out, as_out):
        agg = (agg_ref[0] + agg_ref[1]) * deg_ref[...]
        h0 = h_ref[...]
        pe0 = pe_ref[...]
        u = jnp.maximum(h0 @ w1hr[...] + pe0 @ w1pr[...] + agg @ w1ar[...]
                        + b1r[...], 0.0)
        hn = h0 + u @ w2r[...] + b2r[...]
        p = jnp.maximum(pe0 @ q1pr[...] + agg[:, :_H // 2] @ q1ar[...]
                        + qb1r[...], 0.0)
        pn = pe0 + p @ q2r[...] + qb2r[...]
        h_out[...] = hn
        pe_out[...] = pn
        ad_out[...] = hn @ wdr[...] + pn @ wpdr[...] + bmr[...]
        as_out[...] = hn @ wsr[...] + pn @ wpsr[...]

    args = (h, pe, aggp, degp, w1h, w1p, w1a, b1, w2, b2,
            q1p, q1a, qb1, q2, qb2, wd2, wpd2, ws2, wps2, bm2)
    out = jax.ShapeDtypeStruct((_N, _H), jnp.float32)
    return pl.pallas_call(
        body,
        grid=(_GN,),
        in_specs=[_row_spec(), _row_spec()] + _agg_specs()
        + [_full_spec(a.shape) for a in args[4:]],
        out_specs=[_row_spec()] * 4,
        out_shape=[out] * 4,
    )(*args)


def _tc_update1(h, pe, aggp, degp, w1h, w1p, w1a, b1, w2, b2, nd1, ndb1, nd2, ndb2):
    def body(h_ref, pe_ref, agg_ref, deg_ref, w1hr, w1pr, w1ar, b1r, w2r, b2r,
             nd1r, ndb1r, nd2r, ndb2r, out_ref):
        agg = (agg_ref[0] + agg_ref[1]) * deg_ref[...]
        h0 = h_ref[...]
        u = jnp.maximum(h0 @ w1hr[...] + pe_ref[...] @ w1pr[...]
                        + agg @ w1ar[...] + b1r[...], 0.0)
        hn = h0 + u @ w2r[...] + b2r[...]
        out_ref[...] = jnp.maximum(hn @ nd1r[...] + ndb1r[...], 0.0) @ nd2r[...] \
            + ndb2r[...]

    args = (h, pe, aggp, degp, w1h, w1p, w1a, b1, w2, b2, nd1, ndb1, nd2, ndb2)
    return pl.pallas_call(
        body,
        grid=(_GN,),
        in_specs=[_row_spec(), _row_spec()] + _agg_specs()
        + [_full_spec(a.shape) for a in args[4:]],
        out_specs=_row_spec(),
        out_shape=jax.ShapeDtypeStruct((_N, _H), jnp.float32),
    )(*args)


def _tc_pool_head(nd, batch3d, hw1, hb1, hw2, hb2):
    def body(nd_ref, b_ref, w1, b1, w2, b2, out_ref, acc):
        i = pl.program_id(0)
        ids = b_ref[0, 0, :]
        oh = (ids[:, None]
              == lax.broadcasted_iota(jnp.int32, (_RB, _NG), 1)).astype(jnp.float32)
        contrib = lax.dot_general(oh, nd_ref[...], (((0,), (0,)), ((), ())),
                                  preferred_element_type=jnp.float32)

        @pl.when(i == 0)
        def _():
            acc[...] = contrib

        @pl.when(i > 0)
        def _():
            acc[...] += contrib

        @pl.when(i == _GN - 1)
        def _():
            out_ref[...] = jnp.maximum(acc[...] @ w1[...] + b1[...], 0.0) \
                @ w2[...] + b2[...]

    args = (nd, batch3d, hw1, hb1, hw2, hb2)
    return pl.pallas_call(
        body,
        grid=(_GN,),
        in_specs=[
            _row_spec(),
            pl.BlockSpec((1, 1, _RB), lambda i: (i, 0, 0)),
        ] + [_full_spec(a.shape) for a in args[2:]],
        out_specs=pl.BlockSpec((_NG, _H), lambda i: (0, 0)),
        out_shape=jax.ShapeDtypeStruct((_NG, _H), jnp.float32),
        scratch_shapes=[pltpu.VMEM((_NG, _H), jnp.float32)],
    )(*args)


def kernel(x, pos, pe_init, exW1, exb1, exW2, exb2, epW, epb, mW1, mb1, mW2,
           mb2, eW, eb, unW1, unb1, unW2, unb2, upW1, upb1, upW2, upb2, ndW1,
           ndb1, ndW2, ndb2, hW1, hb1, hW2, hb2, edge_index, batch):
    f32 = jnp.float32
    src = edge_index[0]
    dst = edge_index[1]
    dst2d = dst.reshape(_NB, _W)
    src2d = src.reshape(_NB, _W)
    posx = pos[:, 0]
    posy = pos[:, 1]
    posz = pos[:, 2]
    pe0p = jnp.pad(pe_init, ((0, 0), (0, _H - 16)))
    epWp = jnp.pad(epW, ((0, _H - 16), (0, 0)))

    def r1(b):
        return b.reshape(1, -1)

    wd = [mW1[l, 0:_H] for l in range(2)]
    ws = [mW1[l, _H:2 * _H] for l in range(2)]
    wpd = [mW1[l, 2 * _H:3 * _H] for l in range(2)]
    wps = [mW1[l, 3 * _H:4 * _H] for l in range(2)]
    wdist = [mW1[l, 4 * _H].reshape(1, _H) for l in range(2)]

    sq, degp32 = _sc_sqdist_deg(posx, posy, posz, dst, src)
    degp = _tc_deg_inv(degp32)
    sq3d = sq.reshape(_GE, 1, _EB)
    h0, pe0, ad1, as1 = _tc_embed(
        x, pe0p, exW1, r1(exb1), exW2, r1(exb2), epWp, r1(epb),
        wd[0], wpd[0], ws[0], wps[0], r1(mb1[0]))

    zrow = jnp.zeros((_RPT, _H), f32)

    adg1, asg1 = _sc_gather_pre(ad1, as1, dst2d, src2d)
    msg1 = _tc_edge_mlp(adg1, asg1, sq3d, mW2[0], r1(mb2[0]),
                        eW[0].reshape(1, _H), eb[0].reshape(1, 1), wdist[0])
    aggp1 = _sc_scatter(msg1, dst2d, zrow)
    h1, pe1, ad2, as2 = _tc_update0(
        h0, pe0, aggp1, degp,
        unW1[0, 0:_H], unW1[0, _H:2 * _H], unW1[0, 2 * _H:3 * _H],
        r1(unb1[0]), unW2[0], r1(unb2[0]),
        upW1[0, 0:_H], upW1[0, _H:_H + _H // 2], r1(upb1[0]),
        upW2[0], r1(upb2[0]),
        wd[1], wpd[1], ws[1], wps[1], r1(mb1[1]))

    adg2, asg2 = _sc_gather_pre(ad2, as2, dst2d, src2d)
    msg2 = _tc_edge_mlp(adg2, asg2, sq3d, mW2[1], r1(mb2[1]),
                        eW[1].reshape(1, _H), eb[1].reshape(1, 1), wdist[1])
    aggp2 = _sc_scatter(msg2, dst2d, zrow)
    nd = _tc_update1(
        h1, pe1, aggp2, degp,
        unW1[1, 0:_H], unW1[1, _H:2 * _H], unW1[1, 2 * _H:3 * _H],
        r1(unb1[1]), unW2[1], r1(unb2[1]),
        ndW1, r1(ndb1), ndW2, r1(ndb2))

    batch3d = batch.reshape(_GN, 1, _RB)
    return _tc_pool_head(nd, batch3d, hW1, r1(hb1), hW2, r1(hb2))


# speedup vs baseline: 5.5546x; 1.1064x over previous
import dataclasses
import functools

import jax
import jax.numpy as jnp
from jax import lax
from jax.experimental import pallas as pl
from jax.experimental.pallas import tpu as pltpu
from jax.experimental.pallas import tpu_sc as plsc

_N = 10000
_E = 160000
_H = 128
_NG = 64

_W = 128
_NB = _E // _W
_NBC = _NB // 2
_NP = 10112
_RPT = _NP // 16
_CHUNK = _E // 32
_RB = 400
_GN = _N // _RB
_EB = 640
_GE = _E // _EB


def _vmesh():
    return plsc.VectorSubcoreMesh(core_axis_name="core", subcore_axis_name="subcore")


def _sc_params():
    cp = pltpu.CompilerParams()
    if "needs_layout_passes" in pltpu.CompilerParams.__dataclass_fields__:
        cp = dataclasses.replace(cp, needs_layout_passes=False)
    return cp


def _sc_sqdist_deg(px, py, pz, dst, src):
    ngrp = _CHUNK // 16 + 1

    @functools.partial(
        pl.kernel,
        out_type=(
            jax.ShapeDtypeStruct((_E,), jnp.float32),
            jax.ShapeDtypeStruct((32, _N), jnp.float32),
        ),
        mesh=_vmesh(),
        scratch_types=[
            pltpu.VMEM((_N,), jnp.float32),
            pltpu.VMEM((_N,), jnp.float32),
            pltpu.VMEM((_N,), jnp.float32),
            pltpu.VMEM((_CHUNK + 16,), jnp.int32),
            pltpu.VMEM((_CHUNK + 16,), jnp.int32),
            pltpu.VMEM((_CHUNK + 16,), jnp.float32),
            pltpu.VMEM((_N,), jnp.float32),
        ],
        compiler_params=_sc_params(),
    )
    def k(px_hbm, py_hbm, pz_hbm, d_hbm, s_hbm, o_hbm, deg_hbm,
          px_v, py_v, pz_v, d_v, s_v, o_v, deg_v):
        cid = lax.axis_index("core")
        sid = lax.axis_index("subcore")
        wid = sid * 2 + cid
        base = wid * _CHUNK
        zero16 = jnp.zeros((16,), jnp.int32)
        zero16f = jnp.zeros((16,), jnp.float32)
        one16f = jnp.ones((16,), jnp.float32)
        lane = lax.iota(jnp.int32, 16)
        d_v[pl.ds(_CHUNK - 8, 16)] = zero16
        s_v[pl.ds(_CHUNK - 8, 16)] = zero16

        @pl.loop(0, _N, step=16)
        def _(r):
            deg_v[pl.ds(r, 16)] = zero16f

        pltpu.sync_copy(px_hbm, px_v)
        pltpu.sync_copy(py_hbm, py_v)
        pltpu.sync_copy(pz_hbm, pz_v)
        pltpu.sync_copy(d_hbm.at[pl.ds(base, _CHUNK)], d_v.at[pl.ds(0, _CHUNK)])
        pltpu.sync_copy(s_hbm.at[pl.ds(base, _CHUNK)], s_v.at[pl.ds(0, _CHUNK)])

        @pl.loop(0, ngrp)
        def _(g):
            off = g * 16
            di = d_v[pl.ds(off, 16)]
            si = s_v[pl.ds(off, 16)]
            dx = plsc.load_gather(px_v, [di]) - plsc.load_gather(px_v, [si])
            dy = plsc.load_gather(py_v, [di]) - plsc.load_gather(py_v, [si])
            dz = plsc.load_gather(pz_v, [di]) - plsc.load_gather(pz_v, [si])
            o_v[pl.ds(off, 16)] = dx * dx + dy * dy + dz * dz
            ones = jnp.where(off + lane < _CHUNK, one16f, zero16f)
            plsc.addupdate_scatter(deg_v, [di], ones)

        pltpu.sync_copy(o_v.at[pl.ds(0, _CHUNK)], o_hbm.at[pl.ds(base, _CHUNK)])
        pltpu.sync_copy(deg_v, deg_hbm.at[wid])

    return k(px, py, pz, dst, src)


def _sc_gather_pre(ad, as_, dst2d, src2d):
    @functools.partial(
        pl.kernel,
        out_type=(
            jax.ShapeDtypeStruct((_E, _H), jnp.float32),
            jax.ShapeDtypeStruct((_E, _H), jnp.float32),
        ),
        mesh=_vmesh(),
        scratch_types=[
            pltpu.SemaphoreType.DMA,
            pltpu.SemaphoreType.DMA,
        ],
    )
    def k(ad_hbm, as_hbm, d_hbm, s_hbm, o1_hbm, o2_hbm, sem_a, sem_b):
        def body(d_v, s_v, o1_v, o2_v):
            ca = pltpu.async_copy(ad_hbm.at[d_v.at[0]], o1_v, sem_a)
            cb = pltpu.async_copy(as_hbm.at[s_v.at[0]], o2_v, sem_b)
            ca.wait()
            cb.wait()

        pltpu.emit_pipeline(
            body,
            grid=(_NB,),
            in_specs=[
                pl.BlockSpec((1, _W), lambda i: (i, 0)),
                pl.BlockSpec((1, _W), lambda i: (i, 0)),
            ],
            out_specs=[
                pl.BlockSpec((_W, _H), lambda i: (i, 0)),
                pl.BlockSpec((_W, _H), lambda i: (i, 0)),
            ],
            core_axis_name=("core", "subcore"),
            dimension_semantics=(pltpu.PARALLEL,),
        )(d_hbm, s_hbm, o1_hbm, o2_hbm)

    return k(ad, as_, dst2d, src2d)


def _sc_scatter(msg, dst2d, zrow):
    npt = _NBC // 16 + 1
    assert npt % 2 == 0

    @functools.partial(
        pl.kernel,
        out_type=jax.ShapeDtypeStruct((2, _NP, _H), jnp.float32),
        mesh=_vmesh(),
        scratch_types=[
            pltpu.VMEM_SHARED((_NP, _H), jnp.float32),
            pltpu.VMEM((2, _W, _H), jnp.float32),
            pltpu.VMEM((2, _W), jnp.int32),
            pltpu.SemaphoreType.DMA,
            pltpu.SemaphoreType.DMA,
        ],
    )
    def k(msg_hbm, d_hbm, z_hbm, agg_hbm, shared, m_v, i_v, sem0, sem1):
        cid = lax.axis_index("core")
        sid = lax.axis_index("subcore")
        sems = (sem0, sem1)
        row0 = pl.multiple_of(sid * _RPT, 8)
        pltpu.sync_copy(z_hbm, shared.at[pl.ds(row0, _RPT)])
        plsc.subcore_barrier()

        def start(j, slot):
            b = sid + j * 16

            @pl.when(b < _NBC)
            def _():
                blk = cid * _NBC + b
                pltpu.async_copy(d_hbm.at[blk], i_v.at[slot], sems[slot])
                pltpu.async_copy(msg_hbm.at[pl.ds(blk * _W, _W)], m_v.at[slot],
                                 sems[slot])

        def drain_scatter(j, slot):
            b = sid + j * 16

            @pl.when(b < _NBC)
            def _():
                blk = cid * _NBC + b
                pltpu.make_async_copy(d_hbm.at[blk], i_v.at[slot],
                                      sems[slot]).wait()
                pltpu.make_async_copy(msg_hbm.at[pl.ds(blk * _W, _W)],
                                      m_v.at[slot], sems[slot]).wait()
                pltpu.sync_copy(m_v.at[slot], shared.at[i_v.at[slot]], add=True)

        start(0, 0)

        @pl.loop(0, npt, step=2)
        def _(j):
            start(j + 1, 1)
            drain_scatter(j, 0)
            start(j + 2, 0)
            drain_scatter(j + 1, 1)

        plsc.subcore_barrier()
        pltpu.sync_copy(shared.at[pl.ds(row0, _RPT)],
                        agg_hbm.at[cid].at[pl.ds(row0, _RPT)])

    return k(msg, dst2d, zrow)


def _full_spec(shape):
    n = len(shape)
    return pl.BlockSpec(shape, lambda i, _n=n: (0,) * _n)


def _row_spec():
    return pl.BlockSpec((_RB, _H), lambda i: (i, 0))


def _tc_embed(x, pe0p, exW1, exb1, exW2, exb2, epWp, epb, wd, wpd, ws, wps, bm):
    def body(x_ref, pe0_ref, w1, b1, w2, b2, wp, bp, wdr, wpdr, wsr, wpsr, bmr,
             h_ref, pe_ref, ad_ref, as_ref):
        h1 = jnp.maximum(x_ref[...] @ w1[...] + b1[...], 0.0)
        h = h1 @ w2[...] + b2[...]
        pe = pe0_ref[...] @ wp[...] + bp[...]
        h_ref[...] = h
        pe_ref[...] = pe
        ad_ref[...] = h @ wdr[...] + pe @ wpdr[...] + bmr[...]
        as_ref[...] = h @ wsr[...] + pe @ wpsr[...]

    args = (x, pe0p, exW1, exb1, exW2, exb2, epWp, epb, wd, wpd, ws, wps, bm)
    out = jax.ShapeDtypeStruct((_N, _H), jnp.float32)
    return pl.pallas_call(
        body,
        grid=(_GN,),
        in_specs=[_row_spec(), _row_spec()] + [_full_spec(a.shape) for a in args[2:]],
        out_specs=[_row_spec()] * 4,
        out_shape=[out] * 4,
    )(*args)


def _tc_edge_mlp(adg, asg, sq3d, w2, b2, ew_row, eb11, wdist_row):
    def body(adg_ref, asg_ref, sq_ref, w2r, b2r, ewr, ebr, wdr, out_ref):
        dist = jnp.sqrt(sq_ref[0, 0, :])
        p = adg_ref[...] + asg_ref[...] + dist[:, None] * wdr[...]
        m = jnp.maximum(jnp.maximum(p, 0.0) @ w2r[...] + b2r[...], 0.0)
        g = jax.nn.sigmoid(jnp.sum(m * ewr[...], axis=1, keepdims=True) + ebr[...])
        out_ref[...] = m * g

    args = (adg, asg, sq3d, w2, b2, ew_row, eb11, wdist_row)
    return pl.pallas_call(
        body,
        grid=(_GE,),
        in_specs=[
            pl.BlockSpec((_EB, _H), lambda i: (i, 0)),
            pl.BlockSpec((_EB, _H), lambda i: (i, 0)),
            pl.BlockSpec((1, 1, _EB), lambda i: (i, 0, 0)),
        ] + [_full_spec(a.shape) for a in args[3:]],
        out_specs=pl.BlockSpec((_EB, _H), lambda i: (i, 0)),
        out_shape=jax.ShapeDtypeStruct((_E, _H), jnp.float32),
    )(*args)


def _tc_deg_inv(degp):
    def body(d_ref, out_ref):
        cnt = jnp.sum(d_ref[...], axis=0)
        out_ref[...] = (1.0 / jnp.maximum(cnt, 1.0))[:, None]

    return pl.pallas_call(
        body,
        grid=(1,),
        in_specs=[_full_spec(degp.shape)],
        out_specs=pl.BlockSpec((_N, 1), lambda i: (0, 0)),
        out_shape=jax.ShapeDtypeStruct((_N, 1), jnp.float32),
    )(degp)


def _agg_specs():
    return [
        pl.BlockSpec((2, _RB, _H), lambda i: (0, i, 0)),
        pl.BlockSpec((_RB, 1), lambda i: (i, 0)),
    ]


def _tc_update0(h, pe, aggp, degp, w1h, w1p, w1a, b1, w2, b2,
                q1p, q1a, qb1, q2, qb2, wd2, wpd2, ws2, wps2, bm2):
    def body(h_ref, pe_ref, agg_ref, deg_ref, w1hr, w1pr, w1ar, b1r, w2r, b2r,
             q1pr, q1ar, qb1r, q2r, qb2r, wdr, wpdr, wsr, wpsr, bmr,
             h_out, pe_out, ad_out, as_out):
        agg = (agg_ref[0] + agg_ref[1]) * deg_ref[...]
        h0 = h_ref[...]
        pe0 = pe_ref[...]
        u = jnp.maximum(h0 @ w1hr[...] + pe0 @ w1pr[...] + agg @ w1ar[...]
                        + b1r[...], 0.0)
        hn = h0 + u @ w2r[...] + b2r[...]
        p = jnp.maximum(pe0 @ q1pr[...] + agg[:, :_H // 2] @ q1ar[...]
                        + qb1r[...], 0.0)
        pn = pe0 + p @ q2r[...] + qb2r[...]
        h_out[...] = hn
        pe_out[...] = pn
        ad_out[...] = hn @ wdr[...] + pn @ wpdr[...] + bmr[...]
        as_out[...] = hn @ wsr[...] + pn @ wpsr[...]

    args = (h, pe, aggp, degp, w1h, w1p, w1a, b1, w2, b2,
            q1p, q1a, qb1, q2, qb2, wd2, wpd2, ws2, wps2, bm2)
    out = jax.ShapeDtypeStruct((_N, _H), jnp.float32)
    return pl.pallas_call(
        body,
        grid=(_GN,),
        in_specs=[_row_spec(), _row_spec()] + _agg_specs()
        + [_full_spec(a.shape) for a in args[4:]],
        out_specs=[_row_spec()] * 4,
        out_shape=[out] * 4,
    )(*args)


def _tc_update1(h, pe, aggp, degp, w1h, w1p, w1a, b1, w2, b2, nd1, ndb1, nd2, ndb2):
    def body(h_ref, pe_ref, agg_ref, deg_ref, w1hr, w1pr, w1ar, b1r, w2r, b2r,
             nd1r, ndb1r, nd2r, ndb2r, out_ref):
        agg = (agg_ref[0] + agg_ref[1]) * deg_ref[...]
        h0 = h_ref[...]
        u = jnp.maximum(h0 @ w1hr[...] + pe_ref[...] @ w1pr[...]
                        + agg @ w1ar[...] + b1r[...], 0.0)
        hn = h0 + u @ w2r[...] + b2r[...]
        out_ref[...] = jnp.maximum(hn @ nd1r[...] + ndb1r[...], 0.0) @ nd2r[...] \
            + ndb2r[...]

    args = (h, pe, aggp, degp, w1h, w1p, w1a, b1, w2, b2, nd1, ndb1, nd2, ndb2)
    return pl.pallas_call(
        body,
        grid=(_GN,),
        in_specs=[_row_spec(), _row_spec()] + _agg_specs()
        + [_full_spec(a.shape) for a in args[4:]],
        out_specs=_row_spec(),
        out_shape=jax.ShapeDtypeStruct((_N, _H), jnp.float32),
    )(*args)


def _tc_pool_head(nd, batch3d, hw1, hb1, hw2, hb2):
    def body(nd_ref, b_ref, w1, b1, w2, b2, out_ref, acc):
        i = pl.program_id(0)
        ids = b_ref[0, 0, :]
        oh = (ids[:, None]
              == lax.broadcasted_iota(jnp.int32, (_RB, _NG), 1)).astype(jnp.float32)
        contrib = lax.dot_general(oh, nd_ref[...], (((0,), (0,)), ((), ())),
                                  preferred_element_type=jnp.float32)

        @pl.when(i == 0)
        def _():
            acc[...] = contrib

        @pl.when(i > 0)
        def _():
            acc[...] += contrib

        @pl.when(i == _GN - 1)
        def _():
            out_ref[...] = jnp.maximum(acc[...] @ w1[...] + b1[...], 0.0) \
                @ w2[...] + b2[...]

    args = (nd, batch3d, hw1, hb1, hw2, hb2)
    return pl.pallas_call(
        body,
        grid=(_GN,),
        in_specs=[
            _row_spec(),
            pl.BlockSpec((1, 1, _RB), lambda i: (i, 0, 0)),
        ] + [_full_spec(a.shape) for a in args[2:]],
        out_specs=pl.BlockSpec((_NG, _H), lambda i: (0, 0)),
        out_shape=jax.ShapeDtypeStruct((_NG, _H), jnp.float32),
        scratch_shapes=[pltpu.VMEM((_NG, _H), jnp.float32)],
    )(*args)


def kernel(x, pos, pe_init, exW1, exb1, exW2, exb2, epW, epb, mW1, mb1, mW2,
           mb2, eW, eb, unW1, unb1, unW2, unb2, upW1, upb1, upW2, upb2, ndW1,
           ndb1, ndW2, ndb2, hW1, hb1, hW2, hb2, edge_index, batch):
    f32 = jnp.float32
    src = edge_index[0]
    dst = edge_index[1]
    dst2d = dst.reshape(_NB, _W)
    src2d = src.reshape(_NB, _W)
    posx = pos[:, 0]
    posy = pos[:, 1]
    posz = pos[:, 2]
    pe0p = jnp.pad(pe_init, ((0, 0), (0, _H - 16)))
    epWp = jnp.pad(epW, ((0, _H - 16), (0, 0)))

    def r1(b):
        return b.reshape(1, -1)

    wd = [mW1[l, 0:_H] for l in range(2)]
    ws = [mW1[l, _H:2 * _H] for l in range(2)]
    wpd = [mW1[l, 2 * _H:3 * _H] for l in range(2)]
    wps = [mW1[l, 3 * _H:4 * _H] for l in range(2)]
    wdist = [mW1[l, 4 * _H].reshape(1, _H) for l in range(2)]

    sq, degp32 = _sc_sqdist_deg(posx, posy, posz, dst, src)
    degp = _tc_deg_inv(degp32)
    sq3d = sq.reshape(_GE, 1, _EB)
    h0, pe0, ad1, as1 = _tc_embed(
        x, pe0p, exW1, r1(exb1), exW2, r1(exb2), epWp, r1(epb),
        wd[0], wpd[0], ws[0], wps[0], r1(mb1[0]))

    zrow = jnp.zeros((_RPT, _H), f32)

    adg1, asg1 = _sc_gather_pre(ad1, as1, dst2d, src2d)
    msg1 = _tc_edge_mlp(adg1, asg1, sq3d, mW2[0], r1(mb2[0]),
                        eW[0].reshape(1, _H), eb[0].reshape(1, 1), wdist[0])
    aggp1 = _sc_scatter(msg1, dst2d, zrow)
    h1, pe1, ad2, as2 = _tc_update0(
        h0, pe0, aggp1, degp,
        unW1[0, 0:_H], unW1[0, _H:2 * _H], unW1[0, 2 * _H:3 * _H],
        r1(unb1[0]), unW2[0], r1(unb2[0]),
        upW1[0, 0:_H], upW1[0, _H:_H + _H // 2], r1(upb1[0]),
        upW2[0], r1(upb2[0]),
        wd[1], wpd[1], ws[1], wps[1], r1(mb1[1]))

    adg2, asg2 = _sc_gather_pre(ad2, as2, dst2d, src2d)
    msg2 = _tc_edge_mlp(adg2, asg2, sq3d, mW2[1], r1(mb2[1]),
                        eW[1].reshape(1, _H), eb[1].reshape(1, 1), wdist[1])
    aggp2 = _sc_scatter(msg2, dst2d, zrow)
    nd = _tc_update1(
        h1, pe1, aggp2, degp,
        unW1[1, 0:_H], unW1[1, _H:2 * _H], unW1[1, 2 * _H:3 * _H],
        r1(unb1[1]), unW2[1], r1(unb2[1]),
        ndW1, r1(ndb1), ndW2, r1(ndb2))

    batch3d = batch.reshape(_GN, 1, _RB)
    return _tc_pool_head(nd, batch3d, hW1, r1(hb1), hW2, r1(hb2))


# speedup vs baseline: 5.6594x; 1.0189x over previous
import dataclasses
import functools

import jax
import jax.numpy as jnp
from jax import lax
from jax.experimental import pallas as pl
from jax.experimental.pallas import tpu as pltpu
from jax.experimental.pallas import tpu_sc as plsc

_N = 10000
_E = 160000
_H = 128
_NG = 64

_W = 128
_NB = _E // _W
_NBC = _NB // 2
_NP = 10112
_RPT = _NP // 16
_CHUNK = _E // 32
_RB = 400
_GN = _N // _RB
_EB = 640
_GE = _E // _EB


def _vmesh():
    return plsc.VectorSubcoreMesh(core_axis_name="core", subcore_axis_name="subcore")


def _sc_params():
    cp = pltpu.CompilerParams()
    if "needs_layout_passes" in pltpu.CompilerParams.__dataclass_fields__:
        cp = dataclasses.replace(cp, needs_layout_passes=False)
    return cp


def _sc_sqdist_deg(px, py, pz, dst, src):
    ngrp = _CHUNK // 16 + 1

    @functools.partial(
        pl.kernel,
        out_type=(
            jax.ShapeDtypeStruct((_E,), jnp.float32),
            jax.ShapeDtypeStruct((32, _N), jnp.float32),
        ),
        mesh=_vmesh(),
        scratch_types=[
            pltpu.VMEM((_N,), jnp.float32),
            pltpu.VMEM((_N,), jnp.float32),
            pltpu.VMEM((_N,), jnp.float32),
            pltpu.VMEM((_CHUNK + 16,), jnp.int32),
            pltpu.VMEM((_CHUNK + 16,), jnp.int32),
            pltpu.VMEM((_CHUNK + 16,), jnp.float32),
            pltpu.VMEM((_N,), jnp.float32),
        ],
        compiler_params=_sc_params(),
    )
    def k(px_hbm, py_hbm, pz_hbm, d_hbm, s_hbm, o_hbm, deg_hbm,
          px_v, py_v, pz_v, d_v, s_v, o_v, deg_v):
        cid = lax.axis_index("core")
        sid = lax.axis_index("subcore")
        wid = sid * 2 + cid
        base = wid * _CHUNK
        zero16 = jnp.zeros((16,), jnp.int32)
        zero16f = jnp.zeros((16,), jnp.float32)
        one16f = jnp.ones((16,), jnp.float32)
        lane = lax.iota(jnp.int32, 16)
        d_v[pl.ds(_CHUNK - 8, 16)] = zero16
        s_v[pl.ds(_CHUNK - 8, 16)] = zero16

        @pl.loop(0, _N, step=16)
        def _(r):
            deg_v[pl.ds(r, 16)] = zero16f

        pltpu.sync_copy(px_hbm, px_v)
        pltpu.sync_copy(py_hbm, py_v)
        pltpu.sync_copy(pz_hbm, pz_v)
        pltpu.sync_copy(d_hbm.at[pl.ds(base, _CHUNK)], d_v.at[pl.ds(0, _CHUNK)])
        pltpu.sync_copy(s_hbm.at[pl.ds(base, _CHUNK)], s_v.at[pl.ds(0, _CHUNK)])

        @pl.loop(0, ngrp)
        def _(g):
            off = g * 16
            di = d_v[pl.ds(off, 16)]
            si = s_v[pl.ds(off, 16)]
            dx = plsc.load_gather(px_v, [di]) - plsc.load_gather(px_v, [si])
            dy = plsc.load_gather(py_v, [di]) - plsc.load_gather(py_v, [si])
            dz = plsc.load_gather(pz_v, [di]) - plsc.load_gather(pz_v, [si])
            o_v[pl.ds(off, 16)] = dx * dx + dy * dy + dz * dz
            ones = jnp.where(off + lane < _CHUNK, one16f, zero16f)
            plsc.addupdate_scatter(deg_v, [di], ones)

        pltpu.sync_copy(o_v.at[pl.ds(0, _CHUNK)], o_hbm.at[pl.ds(base, _CHUNK)])
        pltpu.sync_copy(deg_v, deg_hbm.at[wid])

    return k(px, py, pz, dst, src)


def _sc_gather_pre(ad, as_, dst2d, src2d):
    @functools.partial(
        pl.kernel,
        out_type=(
            jax.ShapeDtypeStruct((_E, _H), jnp.float32),
            jax.ShapeDtypeStruct((_E, _H), jnp.float32),
        ),
        mesh=_vmesh(),
        scratch_types=[
            pltpu.SemaphoreType.DMA,
            pltpu.SemaphoreType.DMA,
        ],
    )
    def k(ad_hbm, as_hbm, d_hbm, s_hbm, o1_hbm, o2_hbm, sem_a, sem_b):
        def body(d_v, s_v, o1_v, o2_v):
            ca = pltpu.async_copy(ad_hbm.at[d_v.at[0]], o1_v, sem_a)
            cb = pltpu.async_copy(as_hbm.at[s_v.at[0]], o2_v, sem_b)
            ca.wait()
            cb.wait()

        pltpu.emit_pipeline(
            body,
            grid=(_NB,),
            in_specs=[
                pl.BlockSpec((1, _W), lambda i: (i, 0)),
                pl.BlockSpec((1, _W), lambda i: (i, 0)),
            ],
            out_specs=[
                pl.BlockSpec((_W, _H), lambda i: (i, 0)),
                pl.BlockSpec((_W, _H), lambda i: (i, 0)),
            ],
            core_axis_name=("core", "subcore"),
            dimension_semantics=(pltpu.PARALLEL,),
        )(d_hbm, s_hbm, o1_hbm, o2_hbm)

    return k(ad, as_, dst2d, src2d)


def _sc_scatter(msg, dst2d, zrow):
    npt = _NBC // 16 + 1
    assert npt % 2 == 0

    @functools.partial(
        pl.kernel,
        out_type=jax.ShapeDtypeStruct((2, _NP, _H), jnp.float32),
        mesh=_vmesh(),
        scratch_types=[
            pltpu.VMEM_SHARED((_NP, _H), jnp.float32),
            pltpu.VMEM((2, _W, _H), jnp.float32),
            pltpu.VMEM((2, _W), jnp.int32),
            pltpu.SemaphoreType.DMA,
            pltpu.SemaphoreType.DMA,
        ],
    )
    def k(msg_hbm, d_hbm, z_hbm, agg_hbm, shared, m_v, i_v, sem0, sem1):
        cid = lax.axis_index("core")
        sid = lax.axis_index("subcore")
        sems = (sem0, sem1)
        row0 = pl.multiple_of(sid * _RPT, 8)
        pltpu.sync_copy(z_hbm, shared.at[pl.ds(row0, _RPT)])
        plsc.subcore_barrier()

        def start(j, slot):
            b = sid + j * 16

            @pl.when(b < _NBC)
            def _():
                blk = cid * _NBC + b
                pltpu.async_copy(d_hbm.at[blk], i_v.at[slot], sems[slot])
                pltpu.async_copy(msg_hbm.at[pl.ds(blk * _W, _W)], m_v.at[slot],
                                 sems[slot])

        def drain_scatter(j, slot):
            b = sid + j * 16

            @pl.when(b < _NBC)
            def _():
                blk = cid * _NBC + b
                pltpu.make_async_copy(d_hbm.at[blk], i_v.at[slot],
                                      sems[slot]).wait()
                pltpu.make_async_copy(msg_hbm.at[pl.ds(blk * _W, _W)],
                                      m_v.at[slot], sems[slot]).wait()
                pltpu.sync_copy(m_v.at[slot], shared.at[i_v.at[slot]], add=True)

        start(0, 0)

        @pl.loop(0, npt, step=2)
        def _(j):
            start(j + 1, 1)
            drain_scatter(j, 0)
            start(j + 2, 0)
            drain_scatter(j + 1, 1)

        plsc.subcore_barrier()
        pltpu.sync_copy(shared.at[pl.ds(row0, _RPT)],
                        agg_hbm.at[cid].at[pl.ds(row0, _RPT)])

    return k(msg, dst2d, zrow)


def _full_spec(shape):
    n = len(shape)
    return pl.BlockSpec(shape, lambda i, _n=n: (0,) * _n)


def _row_spec():
    return pl.BlockSpec((_RB, _H), lambda i: (i, 0))


def _tc_embed(x, pe0p, exW1, exb1, exW2, exb2, epWp, epb, wd, wpd, ws, wps, bm):
    def body(x_ref, pe0_ref, w1, b1, w2, b2, wp, bp, wdr, wpdr, wsr, wpsr, bmr,
             h_ref, pe_ref, ad_ref, as_ref):
        h1 = jnp.maximum(x_ref[...] @ w1[...] + b1[...], 0.0)
        h = h1 @ w2[...] + b2[...]
        pe = pe0_ref[...] @ wp[...] + bp[...]
        h_ref[...] = h
        pe_ref[...] = pe
        ad_ref[...] = h @ wdr[...] + pe @ wpdr[...] + bmr[...]
        as_ref[...] = h @ wsr[...] + pe @ wpsr[...]

    args = (x, pe0p, exW1, exb1, exW2, exb2, epWp, epb, wd, wpd, ws, wps, bm)
    out = jax.ShapeDtypeStruct((_N, _H), jnp.float32)
    return pl.pallas_call(
        body,
        grid=(_GN,),
        in_specs=[_row_spec(), _row_spec()] + [_full_spec(a.shape) for a in args[2:]],
        out_specs=[_row_spec()] * 4,
        out_shape=[out] * 4,
    )(*args)


def _tc_edge_mlp(adg, asg, sq3d, w2, b2, ew_row, eb11, wdist_row):
    def body(adg_ref, asg_ref, sq_ref, w2r, b2r, ewr, ebr, wdr, out_ref):
        dist = jnp.sqrt(sq_ref[0, 0, :])
        p = adg_ref[...] + asg_ref[...] + dist[:, None] * wdr[...]
        p = jnp.maximum(p, 0.0).astype(jnp.bfloat16)
        mm = lax.dot_general(p, w2r[...].astype(jnp.bfloat16),
                             (((1,), (0,)), ((), ())),
                             preferred_element_type=jnp.float32)
        m = jnp.maximum(mm + b2r[...], 0.0)
        g = jax.nn.sigmoid(jnp.sum(m * ewr[...], axis=1, keepdims=True) + ebr[...])
        out_ref[...] = m * g

    args = (adg, asg, sq3d, w2, b2, ew_row, eb11, wdist_row)
    return pl.pallas_call(
        body,
        grid=(_GE,),
        in_specs=[
            pl.BlockSpec((_EB, _H), lambda i: (i, 0)),
            pl.BlockSpec((_EB, _H), lambda i: (i, 0)),
            pl.BlockSpec((1, 1, _EB), lambda i: (i, 0, 0)),
        ] + [_full_spec(a.shape) for a in args[3:]],
        out_specs=pl.BlockSpec((_EB, _H), lambda i: (i, 0)),
        out_shape=jax.ShapeDtypeStruct((_E, _H), jnp.float32),
    )(*args)


def _tc_deg_inv(degp):
    def body(d_ref, out_ref):
        cnt = jnp.sum(d_ref[...], axis=0)
        out_ref[...] = (1.0 / jnp.maximum(cnt, 1.0))[:, None]

    return pl.pallas_call(
        body,
        grid=(1,),
        in_specs=[_full_spec(degp.shape)],
        out_specs=pl.BlockSpec((_N, 1), lambda i: (0, 0)),
        out_shape=jax.ShapeDtypeStruct((_N, 1), jnp.float32),
    )(degp)


def _agg_specs():
    return [
        pl.BlockSpec((2, _RB, _H), lambda i: (0, i, 0)),
        pl.BlockSpec((_RB, 1), lambda i: (i, 0)),
    ]


def _tc_update0(h, pe, aggp, degp, w1h, w1p, w1a, b1, w2, b2,
                q1p, q1a, qb1, q2, qb2, wd2, wpd2, ws2, wps2, bm2):
    def body(h_ref, pe_ref, agg_ref, deg_ref, w1hr, w1pr, w1ar, b1r, w2r, b2r,
             q1pr, q1ar, qb1r, q2r, qb2r, wdr, wpdr, wsr, wpsr, bmr,
             h_out, pe_out, ad_out, as_out):
        agg = (agg_ref[0] + agg_ref[1]) * deg_ref[...]
        h0 = h_ref[...]
        pe0 = pe_ref[...]
        u = jnp.maximum(h0 @ w1hr[...] + pe0 @ w1pr[...] + agg @ w1ar[...]
                        + b1r[...], 0.0)
        hn = h0 + u @ w2r[...] + b2r[...]
        p = jnp.maximum(pe0 @ q1pr[...] + agg[:, :_H // 2] @ q1ar[...]
                        + qb1r[...], 0.0)
        pn = pe0 + p @ q2r[...] + qb2r[...]
        h_out[...] = hn
        pe_out[...] = pn
        ad_out[...] = hn @ wdr[...] + pn @ wpdr[...] + bmr[...]
        as_out[...] = hn @ wsr[...] + pn @ wpsr[...]

    args = (h, pe, aggp, degp, w1h, w1p, w1a, b1, w2, b2,
            q1p, q1a, qb1, q2, qb2, wd2, wpd2, ws2, wps2, bm2)
    out = jax.ShapeDtypeStruct((_N, _H), jnp.float32)
    return pl.pallas_call(
        body,
        grid=(_GN,),
        in_specs=[_row_spec(), _row_spec()] + _agg_specs()
        + [_full_spec(a.shape) for a in args[4:]],
        out_specs=[_row_spec()] * 4,
        out_shape=[out] * 4,
    )(*args)


def _tc_update1(h, pe, aggp, degp, batch3d, w1h, w1p, w1a, b1, w2, b2,
                nd1, ndb1, nd2, ndb2, hw1, hb1, hw2, hb2):
    def body(h_ref, pe_ref, agg_ref, deg_ref, b_ref, w1hr, w1pr, w1ar, b1r,
             w2r, b2r, nd1r, ndb1r, nd2r, ndb2r, hw1r, hb1r, hw2r, hb2r,
             out_ref, acc):
        i = pl.program_id(0)
        agg = (agg_ref[0] + agg_ref[1]) * deg_ref[...]
        h0 = h_ref[...]
        u = jnp.maximum(h0 @ w1hr[...] + pe_ref[...] @ w1pr[...]
                        + agg @ w1ar[...] + b1r[...], 0.0)
        hn = h0 + u @ w2r[...] + b2r[...]
        nd = jnp.maximum(hn @ nd1r[...] + ndb1r[...], 0.0) @ nd2r[...] \
            + ndb2r[...]
        ids = b_ref[0, 0, :]
        oh = (ids[:, None]
              == lax.broadcasted_iota(jnp.int32, (_RB, _NG), 1)).astype(jnp.float32)
        contrib = lax.dot_general(oh, nd, (((0,), (0,)), ((), ())),
                                  preferred_element_type=jnp.float32)

        @pl.when(i == 0)
        def _():
            acc[...] = contrib

        @pl.when(i > 0)
        def _():
            acc[...] += contrib

        @pl.when(i == _GN - 1)
        def _():
            out_ref[...] = jnp.maximum(acc[...] @ hw1r[...] + hb1r[...], 0.0) \
                @ hw2r[...] + hb2r[...]

    args = (h, pe, aggp, degp, batch3d, w1h, w1p, w1a, b1, w2, b2,
            nd1, ndb1, nd2, ndb2, hw1, hb1, hw2, hb2)
    return pl.pallas_call(
        body,
        grid=(_GN,),
        in_specs=[_row_spec(), _row_spec()] + _agg_specs()
        + [pl.BlockSpec((1, 1, _RB), lambda i: (i, 0, 0))]
        + [_full_spec(a.shape) for a in args[5:]],
        out_specs=pl.BlockSpec((_NG, _H), lambda i: (0, 0)),
        out_shape=jax.ShapeDtypeStruct((_NG, _H), jnp.float32),
        scratch_shapes=[pltpu.VMEM((_NG, _H), jnp.float32)],
    )(*args)


def kernel(x, pos, pe_init, exW1, exb1, exW2, exb2, epW, epb, mW1, mb1, mW2,
           mb2, eW, eb, unW1, unb1, unW2, unb2, upW1, upb1, upW2, upb2, ndW1,
           ndb1, ndW2, ndb2, hW1, hb1, hW2, hb2, edge_index, batch):
    f32 = jnp.float32
    src = edge_index[0]
    dst = edge_index[1]
    dst2d = dst.reshape(_NB, _W)
    src2d = src.reshape(_NB, _W)
    posx = pos[:, 0]
    posy = pos[:, 1]
    posz = pos[:, 2]
    pe0p = jnp.pad(pe_init, ((0, 0), (0, _H - 16)))
    epWp = jnp.pad(epW, ((0, _H - 16), (0, 0)))

    def r1(b):
        return b.reshape(1, -1)

    wd = [mW1[l, 0:_H] for l in range(2)]
    ws = [mW1[l, _H:2 * _H] for l in range(2)]
    wpd = [mW1[l, 2 * _H:3 * _H] for l in range(2)]
    wps = [mW1[l, 3 * _H:4 * _H] for l in range(2)]
    wdist = [mW1[l, 4 * _H].reshape(1, _H) for l in range(2)]

    sq, degp32 = _sc_sqdist_deg(posx, posy, posz, dst, src)
    degp = _tc_deg_inv(degp32)
    sq3d = sq.reshape(_GE, 1, _EB)
    h0, pe0, ad1, as1 = _tc_embed(
        x, pe0p, exW1, r1(exb1), exW2, r1(exb2), epWp, r1(epb),
        wd[0], wpd[0], ws[0], wps[0], r1(mb1[0]))

    zrow = jnp.zeros((_RPT, _H), f32)

    adg1, asg1 = _sc_gather_pre(ad1, as1, dst2d, src2d)
    msg1 = _tc_edge_mlp(adg1, asg1, sq3d, mW2[0], r1(mb2[0]),
                        eW[0].reshape(1, _H), eb[0].reshape(1, 1), wdist[0])
    aggp1 = _sc_scatter(msg1, dst2d, zrow)
    h1, pe1, ad2, as2 = _tc_update0(
        h0, pe0, aggp1, degp,
        unW1[0, 0:_H], unW1[0, _H:2 * _H], unW1[0, 2 * _H:3 * _H],
        r1(unb1[0]), unW2[0], r1(unb2[0]),
        upW1[0, 0:_H], upW1[0, _H:_H + _H // 2], r1(upb1[0]),
        upW2[0], r1(upb2[0]),
        wd[1], wpd[1], ws[1], wps[1], r1(mb1[1]))

    adg2, asg2 = _sc_gather_pre(ad2, as2, dst2d, src2d)
    msg2 = _tc_edge_mlp(adg2, asg2, sq3d, mW2[1], r1(mb2[1]),
                        eW[1].reshape(1, _H), eb[1].reshape(1, 1), wdist[1])
    aggp2 = _sc_scatter(msg2, dst2d, zrow)
    batch3d = batch.reshape(_GN, 1, _RB)
    return _tc_update1(
        h1, pe1, aggp2, degp, batch3d,
        unW1[1, 0:_H], unW1[1, _H:2 * _H], unW1[1, 2 * _H:3 * _H],
        r1(unb1[1]), unW2[1], r1(unb2[1]),
        ndW1, r1(ndb1), ndW2, r1(ndb2),
        hW1, r1(hb1), hW2, r1(hb2))
